# baseline (speedup 1.0000x reference)
import functools
import math

import jax
import jax.numpy as jnp
from jax.experimental import pallas as pl
from jax.experimental.pallas import tpu as pltpu
from jax.experimental.pallas import tpu_sc as plsc

B = 8
N = 256
D = 128
N_RBF = 20
RBF_PAD = 24
CUTOFF = 5.0
T = 128
LN2 = math.log(2.0)

_width = CUTOFF / (N_RBF - 1)
_COEFF = -0.5 / (_width * _width)
_SCALE = math.sqrt(-_COEFF)


def _ssp(v):
    vc = jnp.minimum(v, 40.0)
    return jnp.log(1.0 + jnp.exp(vc)) - LN2


def _dist(ra, rb):
    dims = (((1,), (1,)), ((), ()))
    prod = jax.lax.dot_general(ra, rb, dims, preferred_element_type=jnp.float32)
    ra2 = jnp.sum(ra * ra, axis=1, keepdims=True)
    rb2 = jax.lax.dot_general(jnp.ones((1, 8), jnp.float32), rb * rb, dims,
                              preferred_element_type=jnp.float32)
    d2 = ra2 + rb2 - 2.0 * prod
    return jnp.sqrt(jnp.maximum(d2, 0.0) + 1e-12)


def _rcut(d):
    return jnp.where(d < CUTOFF,
                     0.5 * (jnp.cos(d * (math.pi / CUTOFF)) + 1.0), 0.0)


def _filter_w3(d, offs_ref, w_f1_ref, b_f1_ref, w_f2_ref, b_f2_ref):
    ds3 = (d * _SCALE).reshape(T, 1, T)
    delta = ds3 - offs_ref[...]
    f3 = jnp.exp(-(delta * delta))
    w1b = jnp.broadcast_to(w_f1_ref[...][None], (T, RBF_PAD, D))
    t13 = jax.lax.dot_general(f3, w1b, (((1,), (1,)), ((0,), (0,))),
                              preferred_element_type=jnp.float32)
    t1 = _ssp(t13.reshape(T * T, D) + b_f1_ref[...])
    w = jnp.dot(t1, w_f2_ref[...], preferred_element_type=jnp.float32)
    return (w + b_f2_ref[...]).reshape(T, T, D)


def _layer_body(x_ref, rp_ref, offs_ref, w_in_ref, b_in_ref,
                w_f1_ref, b_f1_ref, w_f2_ref, b_f2_ref,
                w_o1_ref, b_o1_ref, w_o2_ref, b_o2_ref,
                y_ref, acc_ref, h_ref):
    t = pl.program_id(1)
    fargs = (offs_ref, w_f1_ref, b_f1_ref, w_f2_ref, b_f2_ref)

    @pl.when(t == 0)
    def _packed_diag():
        h_ref[...] = jnp.dot(x_ref[0], w_in_ref[...],
                             preferred_element_type=jnp.float32) + b_in_ref[...]
        r0 = rp_ref[0, 0]
        r1 = rp_ref[0, 1]
        d00 = _dist(r0, r0)
        d11 = _dist(r1, r1)
        pp = jax.lax.broadcasted_iota(jnp.int32, (T, T), 0)
        qq = jax.lax.broadcasted_iota(jnp.int32, (T, T), 1)
        d = jnp.where(qq > pp, d00, d11)
        rc = _rcut(d)
        rc_u = jnp.where(qq > pp, rc, 0.0)
        rc_l = jnp.where(qq < pp, rc, 0.0)
        w3 = _filter_w3(d, *fargs)
        h0 = h_ref[0:T, :]
        h1 = h_ref[T:N, :]
        msg_u = jax.lax.dot_general(rc_u, w3 * h0[None, :, :],
                                    (((1,), (1,)), ((0,), (0,))),
                                    preferred_element_type=jnp.float32)
        msg_l = jnp.sum(w3 * (rc_l[:, :, None] * h1[:, None, :]), axis=0)
        acc_ref[0:T, :] = msg_u
        acc_ref[T:N, :] = msg_l

    @pl.when(t == 1)
    def _offdiag():
        r0 = rp_ref[0, 0]
        r1 = rp_ref[0, 1]
        d = _dist(r0, r1)
        rc = _rcut(d)
        w3 = _filter_w3(d, *fargs)
        h1 = h_ref[T:N, :]
        msg = jax.lax.dot_general(rc, w3 * h1[None, :, :],
                                  (((1,), (1,)), ((0,), (0,))),
                                  preferred_element_type=jnp.float32)
        acc_ref[0:T, :] = acc_ref[0:T, :] + msg
        agg = acc_ref[...]
        o = _ssp(jnp.dot(agg, w_o1_ref[...], preferred_element_type=jnp.float32)
                 + b_o1_ref[...])
        out = jnp.dot(o, w_o2_ref[...], preferred_element_type=jnp.float32) \
            + b_o2_ref[...]
        y_ref[0, :, :] = x_ref[0] + out


def _interaction_layer(x, rp, p, wf1p, offs):
    wspec = pl.BlockSpec((D, D), lambda b, t: (0, 0))
    bspec = pl.BlockSpec((1, D), lambda b, t: (0, 0))
    return pl.pallas_call(
        _layer_body,
        grid=(B, 2),
        in_specs=[
            pl.BlockSpec((1, N, D), lambda b, t: (b, 0, 0)),
            pl.BlockSpec((1, 2, T, 8), lambda b, t: (b, 0, 0, 0)),
            pl.BlockSpec((1, RBF_PAD, 1), lambda b, t: (0, 0, 0)),
            wspec, bspec,
            pl.BlockSpec((RBF_PAD, D), lambda b, t: (0, 0)), bspec,
            wspec, bspec,
            wspec, bspec,
            wspec, bspec,
        ],
        out_specs=pl.BlockSpec((1, N, D), lambda b, t: (b, 0, 0)),
        out_shape=jax.ShapeDtypeStruct((B, N, D), jnp.float32),
        scratch_shapes=[pltpu.VMEM((N, D), jnp.float32),
                        pltpu.VMEM((N, D), jnp.float32)],
    )(x, rp, offs,
      p['w_in'], p['b_in'].reshape(1, D),
      wf1p, p['b_f1'].reshape(1, D),
      p['w_f2'], p['b_f2'].reshape(1, D),
      p['w_o1'], p['b_o1'].reshape(1, D),
      p['w_o2'], p['b_o2'].reshape(1, D))


def _embed_gather_sc(emb, zflat):
    info = plsc.get_sparse_core_info()
    nw = info.num_cores * info.num_subcores
    btot = zflat.shape[0]
    bpw = btot // nw
    mesh = plsc.VectorSubcoreMesh(core_axis_name="c", subcore_axis_name="s")

    @functools.partial(
        pl.kernel, mesh=mesh,
        out_type=jax.ShapeDtypeStruct((btot, D), jnp.float32),
        scratch_types=[
            pltpu.VMEM((bpw,), jnp.int32),
            pltpu.VMEM((bpw, D), jnp.float32),
            pltpu.SemaphoreType.DMA,
        ],
    )
    def gather_k(table_hbm, idx_hbm, out_hbm, idx_v, rows_v, sem):
        wid = jax.lax.axis_index("s") * info.num_cores + jax.lax.axis_index("c")
        base = wid * bpw
        pltpu.sync_copy(idx_hbm.at[pl.ds(base, bpw)], idx_v)
        pltpu.async_copy(table_hbm.at[idx_v], rows_v, sem).wait()
        pltpu.sync_copy(rows_v, out_hbm.at[pl.ds(base, bpw)])

    return gather_k(emb, zflat)


def kernel(Z, R, emb, params):
    x = _embed_gather_sc(emb.astype(jnp.float32),
                         Z.reshape(-1).astype(jnp.int32)).reshape(B, N, D)
    rp = jnp.zeros((B, N, 8), jnp.float32).at[:, :, :3].set(R)
    rp = rp.reshape(B, 2, T, 8)
    ar = jnp.arange(RBF_PAD)
    offs = jnp.where(ar < N_RBF, ar * (_width * _SCALE), 1e6).astype(
        jnp.float32).reshape(1, RBF_PAD, 1)
    for p in params:
        wf1p = jnp.zeros((RBF_PAD, D), jnp.float32).at[:N_RBF].set(p['w_f1'])
        x = _interaction_layer(x, rp, p, wf1p, offs)
    return x


# speedup vs baseline: 1.0023x; 1.0023x over previous
import functools
import math

import jax
import jax.numpy as jnp
from jax.experimental import pallas as pl
from jax.experimental.pallas import tpu as pltpu
from jax.experimental.pallas import tpu_sc as plsc

B = 8
N = 256
D = 128
N_RBF = 20
RBF_PAD = 24
CUTOFF = 5.0
T = 128
LN2 = math.log(2.0)

_width = CUTOFF / (N_RBF - 1)
_COEFF = -0.5 / (_width * _width)
_SCALE = math.sqrt(-_COEFF)


def _ssp(v):
    vc = jnp.minimum(v, 40.0)
    return jnp.log(1.0 + jnp.exp(vc)) - LN2


def _dist(ra, rb):
    dims = (((1,), (1,)), ((), ()))
    prod = jax.lax.dot_general(ra, rb, dims, preferred_element_type=jnp.float32)
    ra2 = jnp.sum(ra * ra, axis=1, keepdims=True)
    rb2 = jax.lax.dot_general(jnp.ones((1, 8), jnp.float32), rb * rb, dims,
                              preferred_element_type=jnp.float32)
    d2 = ra2 + rb2 - 2.0 * prod
    return jnp.sqrt(jnp.maximum(d2, 0.0) + 1e-12)


def _rcut(d):
    return jnp.where(d < CUTOFF,
                     0.5 * (jnp.cos(d * (math.pi / CUTOFF)) + 1.0), 0.0)


def _filter_w3(d, offs_ref, w_f1_ref, b_f1_ref, w_f2_ref, b_f2_ref):
    ds3 = (d * _SCALE).reshape(T, 1, T)
    delta = ds3 - offs_ref[...]
    f3 = jnp.exp(-(delta * delta))
    w1b = jnp.broadcast_to(w_f1_ref[...][None], (T, RBF_PAD, D))
    t13 = jax.lax.dot_general(f3, w1b, (((1,), (1,)), ((0,), (0,))),
                              preferred_element_type=jnp.float32)
    t1 = _ssp(t13.reshape(T * T, D) + b_f1_ref[...])
    w = jnp.dot(t1, w_f2_ref[...], preferred_element_type=jnp.float32)
    return (w + b_f2_ref[...]).reshape(T, T, D)


def _layer_body(x_ref, rp_ref, offs_ref, w_in_ref, b_in_ref,
                w_f1_ref, b_f1_ref, w_f2_ref, b_f2_ref,
                w_o1_ref, b_o1_ref, w_o2_ref, b_o2_ref,
                y_ref, acc_ref, h_ref):
    t = pl.program_id(1)
    fargs = (offs_ref, w_f1_ref, b_f1_ref, w_f2_ref, b_f2_ref)

    @pl.when(t == 0)
    def _packed_diag():
        h_ref[...] = jnp.dot(x_ref[0], w_in_ref[...],
                             preferred_element_type=jnp.float32) + b_in_ref[...]
        r0 = rp_ref[0, 0]
        r1 = rp_ref[0, 1]
        d00 = _dist(r0, r0)
        d11 = _dist(r1, r1)
        pp = jax.lax.broadcasted_iota(jnp.int32, (T, T), 0)
        qq = jax.lax.broadcasted_iota(jnp.int32, (T, T), 1)
        d = jnp.where(qq > pp, d00, d11)
        rc = _rcut(d)
        rc_u = jnp.where(qq > pp, rc, 0.0)
        rc_l = jnp.where(qq < pp, rc, 0.0)
        w3 = _filter_w3(d, *fargs)
        h0 = h_ref[0:T, :]
        h1 = h_ref[T:N, :]
        msg_u = jax.lax.dot_general(rc_u, w3 * h0[None, :, :],
                                    (((1,), (1,)), ((0,), (0,))),
                                    preferred_element_type=jnp.float32)
        msg_l = jnp.sum(w3 * (rc_l[:, :, None] * h1[:, None, :]), axis=0)
        acc_ref[0:T, :] = msg_u
        acc_ref[T:N, :] = msg_l

    @pl.when(t == 1)
    def _offdiag():
        r0 = rp_ref[0, 0]
        r1 = rp_ref[0, 1]
        d = _dist(r0, r1)
        rc = _rcut(d)
        w3 = _filter_w3(d, *fargs)
        h1 = h_ref[T:N, :]
        msg = jax.lax.dot_general(rc, w3 * h1[None, :, :],
                                  (((1,), (1,)), ((0,), (0,))),
                                  preferred_element_type=jnp.float32)
        acc_ref[0:T, :] = acc_ref[0:T, :] + msg
        agg = acc_ref[...]
        o = _ssp(jnp.dot(agg, w_o1_ref[...], preferred_element_type=jnp.float32)
                 + b_o1_ref[...])
        out = jnp.dot(o, w_o2_ref[...], preferred_element_type=jnp.float32) \
            + b_o2_ref[...]
        y_ref[0, :, :] = x_ref[0] + out


def _interaction_layer(x, rp, p, wf1p, offs):
    wspec = pl.BlockSpec((D, D), lambda b, t: (0, 0))
    bspec = pl.BlockSpec((1, D), lambda b, t: (0, 0))
    return pl.pallas_call(
        _layer_body,
        grid=(B, 2),
        in_specs=[
            pl.BlockSpec((1, N, D), lambda b, t: (b, 0, 0)),
            pl.BlockSpec((1, 2, T, 8), lambda b, t: (b, 0, 0, 0)),
            pl.BlockSpec((1, RBF_PAD, 1), lambda b, t: (0, 0, 0)),
            wspec, bspec,
            pl.BlockSpec((RBF_PAD, D), lambda b, t: (0, 0)), bspec,
            wspec, bspec,
            wspec, bspec,
            wspec, bspec,
        ],
        out_specs=pl.BlockSpec((1, N, D), lambda b, t: (b, 0, 0)),
        out_shape=jax.ShapeDtypeStruct((B, N, D), jnp.float32),
        compiler_params=pltpu.CompilerParams(
            dimension_semantics=("parallel", "arbitrary")),
        scratch_shapes=[pltpu.VMEM((N, D), jnp.float32),
                        pltpu.VMEM((N, D), jnp.float32)],
    )(x, rp, offs,
      p['w_in'], p['b_in'].reshape(1, D),
      wf1p, p['b_f1'].reshape(1, D),
      p['w_f2'], p['b_f2'].reshape(1, D),
      p['w_o1'], p['b_o1'].reshape(1, D),
      p['w_o2'], p['b_o2'].reshape(1, D))


def _embed_gather_sc(emb, zflat):
    info = plsc.get_sparse_core_info()
    nw = info.num_cores * info.num_subcores
    btot = zflat.shape[0]
    bpw = btot // nw
    mesh = plsc.VectorSubcoreMesh(core_axis_name="c", subcore_axis_name="s")

    @functools.partial(
        pl.kernel, mesh=mesh,
        out_type=jax.ShapeDtypeStruct((btot, D), jnp.float32),
        scratch_types=[
            pltpu.VMEM((bpw,), jnp.int32),
            pltpu.VMEM((bpw, D), jnp.float32),
            pltpu.SemaphoreType.DMA,
        ],
    )
    def gather_k(table_hbm, idx_hbm, out_hbm, idx_v, rows_v, sem):
        wid = jax.lax.axis_index("s") * info.num_cores + jax.lax.axis_index("c")
        base = wid * bpw
        pltpu.sync_copy(idx_hbm.at[pl.ds(base, bpw)], idx_v)
        pltpu.async_copy(table_hbm.at[idx_v], rows_v, sem).wait()
        pltpu.sync_copy(rows_v, out_hbm.at[pl.ds(base, bpw)])

    return gather_k(emb, zflat)


def kernel(Z, R, emb, params):
    x = _embed_gather_sc(emb.astype(jnp.float32),
                         Z.reshape(-1).astype(jnp.int32)).reshape(B, N, D)
    rp = jnp.zeros((B, N, 8), jnp.float32).at[:, :, :3].set(R)
    rp = rp.reshape(B, 2, T, 8)
    ar = jnp.arange(RBF_PAD)
    offs = jnp.where(ar < N_RBF, ar * (_width * _SCALE), 1e6).astype(
        jnp.float32).reshape(1, RBF_PAD, 1)
    for p in params:
        wf1p = jnp.zeros((RBF_PAD, D), jnp.float32).at[:N_RBF].set(p['w_f1'])
        x = _interaction_layer(x, rp, p, wf1p, offs)
    return x


# speedup vs baseline: 1.0047x; 1.0024x over previous
import functools
import math

import jax
import jax.numpy as jnp
from jax.experimental import pallas as pl
from jax.experimental.pallas import tpu as pltpu
from jax.experimental.pallas import tpu_sc as plsc

B = 8
N = 256
D = 128
N_RBF = 20
RBF_PAD = 24
CUTOFF = 5.0
T = 128
NL = 3
LN2 = math.log(2.0)

_width = CUTOFF / (N_RBF - 1)
_COEFF = -0.5 / (_width * _width)
_SCALE = math.sqrt(-_COEFF)


def _ssp(v):
    vc = jnp.minimum(v, 40.0)
    return jnp.log(1.0 + jnp.exp(vc)) - LN2


def _dist(ra, rb):
    dims = (((1,), (1,)), ((), ()))
    prod = jax.lax.dot_general(ra, rb, dims, preferred_element_type=jnp.float32)
    ra2 = jnp.sum(ra * ra, axis=1, keepdims=True)
    rb2 = jax.lax.dot_general(jnp.ones((1, 8), jnp.float32), rb * rb, dims,
                              preferred_element_type=jnp.float32)
    d2 = ra2 + rb2 - 2.0 * prod
    return jnp.sqrt(jnp.maximum(d2, 0.0) + 1e-12)


def _rcut(d):
    return jnp.where(d < CUTOFF,
                     0.5 * (jnp.cos(d * (math.pi / CUTOFF)) + 1.0), 0.0)


def _filter_w3(d, offs, w_f1, b_f1, w_f2, b_f2):
    ds3 = (d * _SCALE).reshape(T, 1, T)
    delta = ds3 - offs
    f3 = jnp.exp(-(delta * delta))
    w1b = jnp.broadcast_to(w_f1[None], (T, RBF_PAD, D))
    t13 = jax.lax.dot_general(f3, w1b, (((1,), (1,)), ((0,), (0,))),
                              preferred_element_type=jnp.float32)
    t1 = _ssp(t13.reshape(T * T, D) + b_f1)
    w = jnp.dot(t1, w_f2, preferred_element_type=jnp.float32)
    return (w + b_f2).reshape(T, T, D)


def _layer_body(x_ref, rp_ref, offs_ref, w_in_ref, b_in_ref,
                w_f1_ref, b_f1_ref, w_f2_ref, b_f2_ref,
                w_o1_ref, b_o1_ref, w_o2_ref, b_o2_ref,
                y_ref, acc_ref, h_ref, xs_ref):
    l = pl.program_id(0)
    b = pl.program_id(1)
    t = pl.program_id(2)
    fargs = (offs_ref[...], w_f1_ref[0], b_f1_ref[0], w_f2_ref[0], b_f2_ref[0])
    xb = pl.ds(b * N, N)

    @pl.when(t == 0)
    def _packed_diag():
        @pl.when(l == 0)
        def _seed():
            xs_ref[xb, :] = x_ref[0]

        h_ref[...] = jnp.dot(xs_ref[xb, :], w_in_ref[0],
                             preferred_element_type=jnp.float32) + b_in_ref[0]
        r0 = rp_ref[0, 0]
        r1 = rp_ref[0, 1]
        d00 = _dist(r0, r0)
        d11 = _dist(r1, r1)
        pp = jax.lax.broadcasted_iota(jnp.int32, (T, T), 0)
        qq = jax.lax.broadcasted_iota(jnp.int32, (T, T), 1)
        d = jnp.where(qq > pp, d00, d11)
        rc = _rcut(d)
        rc_u = jnp.where(qq > pp, rc, 0.0)
        rc_l = jnp.where(qq < pp, rc, 0.0)
        w3 = _filter_w3(d, *fargs)
        h0 = h_ref[0:T, :]
        h1 = h_ref[T:N, :]
        msg_u = jax.lax.dot_general(rc_u, w3 * h0[None, :, :],
                                    (((1,), (1,)), ((0,), (0,))),
                                    preferred_element_type=jnp.float32)
        msg_l = jnp.sum(w3 * (rc_l[:, :, None] * h1[:, None, :]), axis=0)
        acc_ref[0:T, :] = msg_u
        acc_ref[T:N, :] = msg_l

    @pl.when(t == 1)
    def _offdiag():
        r0 = rp_ref[0, 0]
        r1 = rp_ref[0, 1]
        d = _dist(r0, r1)
        rc = _rcut(d)
        w3 = _filter_w3(d, *fargs)
        h1 = h_ref[T:N, :]
        msg = jax.lax.dot_general(rc, w3 * h1[None, :, :],
                                  (((1,), (1,)), ((0,), (0,))),
                                  preferred_element_type=jnp.float32)
        acc_ref[0:T, :] = acc_ref[0:T, :] + msg
        agg = acc_ref[...]
        o = _ssp(jnp.dot(agg, w_o1_ref[0], preferred_element_type=jnp.float32)
                 + b_o1_ref[0])
        out = jnp.dot(o, w_o2_ref[0], preferred_element_type=jnp.float32) \
            + b_o2_ref[0]
        xnew = xs_ref[xb, :] + out
        xs_ref[xb, :] = xnew

        @pl.when(l == NL - 1)
        def _emit():
            y_ref[0, :, :] = xnew


def _interactions(x, rp, ws, offs):
    wspec = pl.BlockSpec((1, D, D), lambda l, b, t: (l, 0, 0))
    bspec = pl.BlockSpec((1, 1, D), lambda l, b, t: (l, 0, 0))
    return pl.pallas_call(
        _layer_body,
        grid=(NL, B, 2),
        in_specs=[
            pl.BlockSpec((1, N, D), lambda l, b, t: (b, 0, 0)),
            pl.BlockSpec((1, 2, T, 8), lambda l, b, t: (b, 0, 0, 0)),
            pl.BlockSpec((1, RBF_PAD, 1), lambda l, b, t: (0, 0, 0)),
            wspec, bspec,
            pl.BlockSpec((1, RBF_PAD, D), lambda l, b, t: (l, 0, 0)), bspec,
            wspec, bspec,
            wspec, bspec,
            wspec, bspec,
        ],
        out_specs=pl.BlockSpec((1, N, D), lambda l, b, t: (b, 0, 0)),
        out_shape=jax.ShapeDtypeStruct((B, N, D), jnp.float32),
        scratch_shapes=[pltpu.VMEM((N, D), jnp.float32),
                        pltpu.VMEM((N, D), jnp.float32),
                        pltpu.VMEM((B * N, D), jnp.float32)],
    )(x, rp, offs, *ws)


def _embed_gather_sc(emb, zflat):
    info = plsc.get_sparse_core_info()
    nw = info.num_cores * info.num_subcores
    btot = zflat.shape[0]
    bpw = btot // nw
    mesh = plsc.VectorSubcoreMesh(core_axis_name="c", subcore_axis_name="s")

    @functools.partial(
        pl.kernel, mesh=mesh,
        out_type=jax.ShapeDtypeStruct((btot, D), jnp.float32),
        scratch_types=[
            pltpu.VMEM((bpw,), jnp.int32),
            pltpu.VMEM((bpw, D), jnp.float32),
            pltpu.SemaphoreType.DMA,
        ],
    )
    def gather_k(table_hbm, idx_hbm, out_hbm, idx_v, rows_v, sem):
        wid = jax.lax.axis_index("s") * info.num_cores + jax.lax.axis_index("c")
        base = wid * bpw
        pltpu.sync_copy(idx_hbm.at[pl.ds(base, bpw)], idx_v)
        pltpu.async_copy(table_hbm.at[idx_v], rows_v, sem).wait()
        pltpu.sync_copy(rows_v, out_hbm.at[pl.ds(base, bpw)])

    return gather_k(emb, zflat)


def kernel(Z, R, emb, params):
    x = _embed_gather_sc(emb.astype(jnp.float32),
                         Z.reshape(-1).astype(jnp.int32)).reshape(B, N, D)
    rp = jnp.zeros((B, N, 8), jnp.float32).at[:, :, :3].set(R)
    rp = rp.reshape(B, 2, T, 8)
    ar = jnp.arange(RBF_PAD)
    offs = jnp.where(ar < N_RBF, ar * (_width * _SCALE), 1e6).astype(
        jnp.float32).reshape(1, RBF_PAD, 1)
    stack = lambda k: jnp.stack([p[k] for p in params])
    bstack = lambda k: jnp.stack([p[k].reshape(1, D) for p in params])
    wf1p = jnp.stack([
        jnp.zeros((RBF_PAD, D), jnp.float32).at[:N_RBF].set(p['w_f1'])
        for p in params])
    ws = (stack('w_in'), bstack('b_in'), wf1p, bstack('b_f1'),
          stack('w_f2'), bstack('b_f2'), stack('w_o1'), bstack('b_o1'),
          stack('w_o2'), bstack('b_o2'))
    return _interactions(x, rp, ws, offs)
